# baseline (speedup 1.0000x reference)
import functools

import jax
import jax.numpy as jnp
from jax import lax
from jax.experimental import pallas as pl
from jax.experimental.pallas import tpu as pltpu
from jax.experimental.pallas import tpu_sc as plsc

_SCALE_MULT = 0.05
_GRID = 0.005
_RADII_MIN = 1.0

_CPA = 8
_NCELLS = _CPA ** 3
_CAP = 28
_MARGIN = 0.06
_NW = 32
_OSTRIDE = 32
_NF = 21

_NT = 6144
_TC_BLK = 1024


def _prep_kernel(means_ref, meansT_ref, rotT_ref, scalesT_ref, semT_ref,
                 semv_ref, ucol_ref, opas_ref, u_ref, mintT_ref, radii_ref,
                 attrs_ref, attrsi_ref, counts_ref, slots_ref,
                 r0T_ref, r1T_ref, r2T_ref, ck_ref, w_ref, B_ref):
    f32, i32 = jnp.float32, jnp.int32
    rotT = rotT_ref[...]
    w = rotT[0:1, :]
    x = rotT[1:2, :]
    y = rotT[2:3, :]
    z = rotT[3:4, :]
    norm = jnp.sqrt(w * w + x * x + y * y + z * z)
    w = w / norm
    x = x / norm
    y = y / norm
    z = z / norm
    rcol = (
        (1 - 2 * (y * y + z * z), 2 * (x * y + w * z), 2 * (x * z - w * y)),
        (2 * (x * y - w * z), 1 - 2 * (x * x + z * z), 2 * (y * z + w * x)),
        (2 * (x * z + w * y), 2 * (y * z - w * x), 1 - 2 * (x * x + y * y)),
    )
    mT = meansT_ref[...]
    sT = scalesT_ref[...]

    rnd = lambda t: t.astype(jnp.bfloat16).astype(f32)
    bits = lambda t: jax.lax.bitcast_convert_type(t, i32)
    packpair = lambda a, b: ((bits(rnd(a)) & ~0xFFFF)
                             | lax.shift_right_logical(bits(rnd(b)), 16))

    for k, rT_ref in enumerate((r0T_ref, r1T_ref, r2T_ref)):
        rT_ref[...] = jnp.concatenate(
            [rcol[k][0], rcol[k][1], rcol[k][2]], axis=0).astype(jnp.bfloat16)
    cks = []
    for k in range(3):
        r0, r1, r2 = rcol[k]
        cks.append(mT[0:1, :] * r0 + mT[1:2, :] * r1 + mT[2:3, :] * r2)
    ck_ref[...] = jnp.concatenate(cks, axis=0)
    wks = [1.0 / (sT[k:k + 1, :] * sT[k:k + 1, :] + 1e-8) for k in range(3)]
    w_ref[...] = jnp.concatenate(wks, axis=0)
    semT = semT_ref[...]
    bfields = [semT[j:j + 1, :] for j in range(semT.shape[0])] + [u_ref[...]]
    B_ref[...] = jnp.concatenate(
        [semv_ref[...], ucol_ref[...], jnp.ones_like(ucol_ref[...])],
        axis=1).astype(jnp.bfloat16)

    rflat = [rcol[k][i] for k in range(3) for i in range(3)]
    words = []
    for j in range(4):
        words.append(packpair(rflat[2 * j], rflat[2 * j + 1]))
    words.append(packpair(rflat[8], jnp.zeros_like(rflat[8])))
    for j in range(9):
        words.append(packpair(bfields[2 * j], bfields[2 * j + 1]))
    for k in range(3):
        words.append(bits(wks[k]))
    for k in range(3):
        words.append(bits(cks[k]))
    words.append(bits(opas_ref[...]))
    attrs_ref[...] = jnp.concatenate(words, axis=0)

    mintT = mintT_ref[...]
    radii = radii_ref[...]
    attrsi_ref[...] = ((radii << 24) | (mintT[0:1, :] << 16)
                       | (mintT[1:2, :] << 8) | mintT[2:3, :])

    gx = (mT[0:1, :] * float(_CPA) + 1.0).astype(i32) - 1
    gy = (mT[1:2, :] * float(_CPA) + 1.0).astype(i32) - 1
    gz = (mT[2:3, :] * float(_CPA) + 1.0).astype(i32) - 1
    cell = (gx * _CPA + gy) * _CPA + gz

    mc = means_ref[...]
    gxc = (mc[:, 0:1] * float(_CPA) + 1.0).astype(i32) - 1
    gyc = (mc[:, 1:2] * float(_CPA) + 1.0).astype(i32) - 1
    gzc = (mc[:, 2:3] * float(_CPA) + 1.0).astype(i32) - 1
    cellcol = (gxc * _CPA + gyc) * _CPA + gzc

    m = cell.shape[1]
    cells_iota = lax.broadcasted_iota(i32, (_NCELLS, m), 0)
    A = (cells_iota == cell).astype(f32)
    counts = jnp.sum(A, axis=1, keepdims=True)
    counts_ref[...] = jnp.minimum(counts, float(_CAP)).astype(i32)
    eq = cellcol == cell
    mi = lax.broadcasted_iota(i32, (m, m), 0)
    mj = lax.broadcasted_iota(i32, (m, m), 1)
    E = (eq & (mj < mi)).astype(f32)
    rank = jnp.sum(E, axis=1, keepdims=True).astype(i32)
    k_iota = lax.broadcasted_iota(i32, (m, _CAP), 1)
    gid1 = (lax.broadcasted_iota(i32, (m, _CAP), 0) + 1).astype(f32)
    W = jnp.where(rank == k_iota, gid1, 0.0)
    slots = jax.lax.dot_general(
        A, W, (((1,), (0,)), ((), ())),
        precision=jax.lax.Precision.HIGHEST,
        preferred_element_type=f32)
    slots_ref[...] = slots.astype(i32)


def _agg_kernel(pts_ref, pint_ref, r0_ref, r1_ref, r2_ref, ck_ref, w_ref,
                opas_ref, mint_ref, radii_ref, B_ref, out_ref):
    pts = pts_ref[...]
    ck = ck_ref[...]
    w = w_ref[...]

    power = None
    for k, r_ref in enumerate((r0_ref, r1_ref, r2_ref)):
        Pk = jnp.dot(pts, r_ref[...], preferred_element_type=jnp.float32)
        dd = Pk - ck[k:k + 1, :]
        term = dd * dd * w[k:k + 1, :]
        power = term if power is None else power + term

    pint = pint_ref[...]
    mint = mint_ref[...]
    radii = radii_ref[...]
    mask = (
        (jnp.abs(pint[:, 0:1] - mint[0:1, :]) <= radii)
        & (jnp.abs(pint[:, 1:2] - mint[1:2, :]) <= radii)
        & (jnp.abs(pint[:, 2:3] - mint[2:3, :]) <= radii)
    )

    g = jnp.exp(-0.5 * power) * opas_ref[...]
    a = jnp.where(mask, g, 0.0).astype(jnp.bfloat16)
    out_ref[...] = jnp.dot(a, B_ref[...], preferred_element_type=jnp.float32)


def _bf16_round(t):
    yv = plsc.bitcast(t, jnp.int32)
    r = (yv + 0x7FFF + ((yv >> 16) & 1)) & ~0xFFFF
    return plsc.bitcast(r, jnp.float32)


def _hi(word):
    return plsc.bitcast(word & ~0xFFFF, jnp.float32)


def _lo(word):
    return plsc.bitcast(word << 16, jnp.float32)


def _ptprep_kernel(npad, pts_ref, pint_ref,
                   px_ref, py_ref, pz_ref, ppk_ref):
    ns = pts_ref.shape[0]
    pad = npad - ns
    rnd = lambda t: t.astype(jnp.bfloat16).astype(jnp.float32)
    for k, ref in enumerate((px_ref, py_ref, pz_ref)):
        ref[...] = jnp.concatenate(
            [rnd(pts_ref[:, k:k + 1]), jnp.full((pad, 1), 10.0, jnp.float32)],
            axis=0)
    pint = pint_ref[...]
    pk = (pint[:, 0:1] << 20) | (pint[:, 1:2] << 10) | pint[:, 2:3]
    ppk_ref[...] = jnp.concatenate([pk, jnp.zeros((pad, 1), jnp.int32)], axis=0)


def _sc_body(m, chunk, attrs_hbm, attrsi_hbm, counts_hbm, slots_hbm,
             px_hbm, py_hbm, pz_hbm, ppk_hbm, out_hbm,
             attrs_v, attrsi_v, counts_v, slots_v,
             px_v, py_v, pz_v, ppk_v, out_v):
    wid = lax.axis_index("s") * 2 + lax.axis_index("c")
    base = wid * chunk
    pltpu.sync_copy(attrs_hbm, attrs_v)
    pltpu.sync_copy(attrsi_hbm, attrsi_v)
    pltpu.sync_copy(counts_hbm, counts_v)
    pltpu.sync_copy(slots_hbm, slots_v)
    pltpu.sync_copy(px_hbm.at[pl.ds(base, chunk)], px_v)
    pltpu.sync_copy(py_hbm.at[pl.ds(base, chunk)], py_v)
    pltpu.sync_copy(pz_hbm.at[pl.ds(base, chunk)], pz_v)
    pltpu.sync_copy(ppk_hbm.at[pl.ds(base, chunk)], ppk_v)

    zeros16 = jnp.zeros((16,), jnp.float32)

    def zero_body(i, carry):
        out_v[pl.ds(i * 16, 16)] = zeros16
        return carry

    lax.fori_loop(0, chunk * _OSTRIDE // 16, zero_body, 0)

    lane = lax.iota(jnp.int32, 16)

    def vreg_body(i, carry):
        sl = pl.ds(i * 16, 16)
        px = px_v[sl]
        py = py_v[sl]
        pz = pz_v[sl]
        ppk = ppk_v[sl]
        pxi = ppk >> 20
        pyi = (ppk >> 10) & 1023
        pzi = ppk & 1023
        obase = (i * 16 + lane) * _OSTRIDE

        los, his = [], []
        for coord in (px, py, pz):
            lo_f = (coord - _MARGIN) * float(_CPA)
            hi_f = (coord + _MARGIN) * float(_CPA)
            los.append((lo_f + 1.0).astype(jnp.int32) - 1)
            his.append((hi_f + 1.0).astype(jnp.int32) - 1)

        def visit(cell, valid):
            cellc = jnp.where(valid, cell, 0)
            cnt = plsc.load_gather(counts_v, [cellc], mask=valid)
            cnt = jnp.where(valid, cnt, 0)
            mx = jnp.max(cnt)

            def jbody(j, jcarry):
                act = j < cnt
                sidx = jnp.where(act, cellc * _CAP + j, 0)
                gid1 = plsc.load_gather(slots_v, [sidx], mask=act)
                act2 = act & (gid1 > 0)
                gid = jnp.where(act2, gid1 - 1, 0)
                pk = plsc.load_gather(attrsi_v, [gid], mask=act2)
                rad = pk >> 24
                mxc = (pk >> 16) & 255
                myc = (pk >> 8) & 255
                mzc = pk & 255
                hit = (act2
                       & (jnp.abs(pxi - mxc) <= rad)
                       & (jnp.abs(pyi - myc) <= rad)
                       & (jnp.abs(pzi - mzc) <= rad))
                any_hit = jnp.max(hit.astype(jnp.int32))

                @pl.when(any_hit > 0)
                def _eval():
                    rw = [plsc.load_gather(attrs_v, [j * m + gid], mask=hit)
                          for j in range(5)]
                    r00, r01 = _hi(rw[0]), _lo(rw[0])
                    r02, r10 = _hi(rw[1]), _lo(rw[1])
                    r11, r12 = _hi(rw[2]), _lo(rw[2])
                    r20, r21 = _hi(rw[3]), _lo(rw[3])
                    r22 = _hi(rw[4])
                    rc = ((r00, r01, r02), (r10, r11, r12), (r20, r21, r22))
                    power = None
                    for k in range(3):
                        wk = plsc.bitcast(
                            plsc.load_gather(attrs_v, [(14 + k) * m + gid], mask=hit),
                            jnp.float32)
                        ck = plsc.bitcast(
                            plsc.load_gather(attrs_v, [(17 + k) * m + gid], mask=hit),
                            jnp.float32)
                        dd = px * rc[k][0] + py * rc[k][1] + pz * rc[k][2] - ck
                        t = dd * dd * wk
                        power = t if power is None else power + t
                    opa = plsc.bitcast(
                        plsc.load_gather(attrs_v, [20 * m + gid], mask=hit),
                        jnp.float32)
                    a = jnp.exp(power * -0.5) * opa
                    a_r = _bf16_round(jnp.where(hit, a, 0.0))
                    plsc.addupdate_scatter(out_v, [obase + 18], a_r, mask=hit)
                    for j in range(9):
                        bw = plsc.load_gather(attrs_v, [(5 + j) * m + gid], mask=hit)
                        plsc.addupdate_scatter(out_v, [obase + 2 * j],
                                               a_r * _hi(bw), mask=hit)
                        plsc.addupdate_scatter(out_v, [obase + 2 * j + 1],
                                               a_r * _lo(bw), mask=hit)

                return jcarry

            lax.fori_loop(0, mx, jbody, 0)

        for dx in range(2):
            cx = los[0] + dx
            mvx = (cx >= 0) & (cx <= _CPA - 1)
            if dx == 1:
                mvx = mvx & (his[0] >= cx)
            for dy in range(2):
                cy = los[1] + dy
                mvy = mvx & (cy >= 0) & (cy <= _CPA - 1)
                if dy == 1:
                    mvy = mvy & (his[1] >= cy)
                for dz in range(2):
                    cz = los[2] + dz
                    mvz = mvy & (cz >= 0) & (cz <= _CPA - 1)
                    if dz == 1:
                        mvz = mvz & (his[2] >= cz)
                    visit((cx * _CPA + cy) * _CPA + cz, mvz)
        return carry

    lax.fori_loop(0, chunk // 16, vreg_body, 0)
    pltpu.sync_copy(out_v, out_hbm.at[pl.ds(base * _OSTRIDE, chunk * _OSTRIDE)])


def kernel(pts, means3D, opas, u, v, semantics, scales, rot3D):
    pts = pts[0]
    means3D = means3D[0]
    opas = opas[0]
    u = u[0]
    v = v[0]
    semantics = semantics[0]
    scales = scales[0]
    rot3D = rot3D[0]

    n, m = pts.shape[0], means3D.shape[0]
    c = semantics.shape[1]

    ns = n - _NT
    chunk = -(-ns // (_NW * 16)) * 16
    npad = _NW * chunk

    semv = semantics * v[:, None]
    pint = jnp.floor(pts / _GRID).astype(jnp.int32)
    mintT = jnp.floor(means3D / _GRID).astype(jnp.int32).T
    radii = jnp.maximum(
        jnp.ceil(scales.max(axis=-1) * _SCALE_MULT / _GRID), _RADII_MIN
    ).astype(jnp.int32)[None, :]

    (attrs, attrsi, counts, slots,
     r0T, r1T, r2T, cks, ws, B) = pl.pallas_call(
        _prep_kernel,
        out_shape=[
            jax.ShapeDtypeStruct((_NF, m), jnp.int32),
            jax.ShapeDtypeStruct((1, m), jnp.int32),
            jax.ShapeDtypeStruct((_NCELLS, 1), jnp.int32),
            jax.ShapeDtypeStruct((_NCELLS, _CAP), jnp.int32),
            jax.ShapeDtypeStruct((3, m), jnp.bfloat16),
            jax.ShapeDtypeStruct((3, m), jnp.bfloat16),
            jax.ShapeDtypeStruct((3, m), jnp.bfloat16),
            jax.ShapeDtypeStruct((3, m), jnp.float32),
            jax.ShapeDtypeStruct((3, m), jnp.float32),
            jax.ShapeDtypeStruct((m, c + 2), jnp.bfloat16),
        ],
    )(means3D, means3D.T, rot3D.T, scales.T, semv.T, semv, u[:, None],
      opas[None, :], u[None, :], mintT, radii)

    pxr, pyr, pzr, ppk = pl.pallas_call(
        functools.partial(_ptprep_kernel, npad),
        out_shape=[
            jax.ShapeDtypeStruct((npad, 1), jnp.float32),
            jax.ShapeDtypeStruct((npad, 1), jnp.float32),
            jax.ShapeDtypeStruct((npad, 1), jnp.float32),
            jax.ShapeDtypeStruct((npad, 1), jnp.int32),
        ],
    )(pts[_NT:], pint[_NT:])

    sc = pl.kernel(
        functools.partial(_sc_body, m, chunk),
        out_type=jax.ShapeDtypeStruct((npad * _OSTRIDE,), jnp.float32),
        mesh=plsc.VectorSubcoreMesh(core_axis_name="c", subcore_axis_name="s"),
        compiler_params=pltpu.CompilerParams(needs_layout_passes=False),
        scratch_types=[
            pltpu.VMEM((_NF * m,), jnp.int32),
            pltpu.VMEM((m,), jnp.int32),
            pltpu.VMEM((_NCELLS,), jnp.int32),
            pltpu.VMEM((_NCELLS * _CAP,), jnp.int32),
            pltpu.VMEM((chunk,), jnp.float32),
            pltpu.VMEM((chunk,), jnp.float32),
            pltpu.VMEM((chunk,), jnp.float32),
            pltpu.VMEM((chunk,), jnp.int32),
            pltpu.VMEM((chunk * _OSTRIDE,), jnp.float32),
        ],
    )
    out_s = sc(attrs.reshape(-1), attrsi.reshape(-1), counts.reshape(-1),
               slots.reshape(-1), pxr.reshape(-1), pyr.reshape(-1),
               pzr.reshape(-1), ppk.reshape(-1))
    out_s = out_s.reshape(npad, _OSTRIDE)

    grid = _NT // _TC_BLK
    full = lambda i: (0, 0)
    out_d = pl.pallas_call(
        _agg_kernel,
        grid=(grid,),
        in_specs=[
            pl.BlockSpec((_TC_BLK, 3), lambda i: (i, 0)),
            pl.BlockSpec((_TC_BLK, 3), lambda i: (i, 0)),
            pl.BlockSpec((3, m), full),
            pl.BlockSpec((3, m), full),
            pl.BlockSpec((3, m), full),
            pl.BlockSpec((3, m), full),
            pl.BlockSpec((3, m), full),
            pl.BlockSpec((1, m), full),
            pl.BlockSpec((3, m), full),
            pl.BlockSpec((1, m), full),
            pl.BlockSpec((m, c + 2), full),
        ],
        out_specs=pl.BlockSpec((_TC_BLK, c + 2), lambda i: (i, 0)),
        out_shape=jax.ShapeDtypeStruct((_NT, c + 2), jnp.float32),
        compiler_params=pltpu.CompilerParams(
            dimension_semantics=("arbitrary",),
        ),
    )(pts[:_NT].astype(jnp.bfloat16), pint[:_NT], r0T, r1T, r2T,
      cks, ws, opas[None, :], mintT, radii, B)

    logits = jnp.concatenate([out_d[:, :c], out_s[:ns, :c]], axis=0)
    bin_logits = jnp.concatenate([out_d[:, c], out_s[:ns, c]], axis=0)
    density = jnp.concatenate([out_d[:, c + 1], out_s[:ns, c + 1]], axis=0)
    return logits, bin_logits, density


# speedup vs baseline: 1.1035x; 1.1035x over previous
import functools

import jax
import jax.numpy as jnp
from jax import lax
from jax.experimental import pallas as pl
from jax.experimental.pallas import tpu as pltpu
from jax.experimental.pallas import tpu_sc as plsc

_SCALE_MULT = 0.05
_GRID = 0.005
_RADII_MIN = 1.0

_CPA = 8
_NCELLS = _CPA ** 3
_CAP = 28
_MARGIN = 0.06
_NW = 32
_OSTRIDE = 32
_NF = 21

_NT = 6144
_TC_BLK = 1024


def _prep_kernel(means_ref, meansT_ref, rotT_ref, scalesT_ref, semT_ref,
                 semv_ref, ucol_ref, opas_ref, u_ref, mintT_ref, radii_ref,
                 attrs_ref, attrsi_ref, counts_ref, slots_ref,
                 r0T_ref, r1T_ref, r2T_ref, ck_ref, w_ref, B_ref):
    f32, i32 = jnp.float32, jnp.int32
    rotT = rotT_ref[...]
    w = rotT[0:1, :]
    x = rotT[1:2, :]
    y = rotT[2:3, :]
    z = rotT[3:4, :]
    norm = jnp.sqrt(w * w + x * x + y * y + z * z)
    w = w / norm
    x = x / norm
    y = y / norm
    z = z / norm
    rcol = (
        (1 - 2 * (y * y + z * z), 2 * (x * y + w * z), 2 * (x * z - w * y)),
        (2 * (x * y - w * z), 1 - 2 * (x * x + z * z), 2 * (y * z + w * x)),
        (2 * (x * z + w * y), 2 * (y * z - w * x), 1 - 2 * (x * x + y * y)),
    )
    mT = meansT_ref[...]
    sT = scalesT_ref[...]

    rnd = lambda t: t.astype(jnp.bfloat16).astype(f32)
    bits = lambda t: jax.lax.bitcast_convert_type(t, i32)
    packpair = lambda a, b: ((bits(rnd(a)) & ~0xFFFF)
                             | lax.shift_right_logical(bits(rnd(b)), 16))

    for k, rT_ref in enumerate((r0T_ref, r1T_ref, r2T_ref)):
        rT_ref[...] = jnp.concatenate(
            [rcol[k][0], rcol[k][1], rcol[k][2]], axis=0).astype(jnp.bfloat16)
    cks = []
    for k in range(3):
        r0, r1, r2 = rcol[k]
        cks.append(mT[0:1, :] * r0 + mT[1:2, :] * r1 + mT[2:3, :] * r2)
    ck_ref[...] = jnp.concatenate(cks, axis=0)
    wks = [1.0 / (sT[k:k + 1, :] * sT[k:k + 1, :] + 1e-8) for k in range(3)]
    w_ref[...] = jnp.concatenate(wks, axis=0)
    semT = semT_ref[...]
    bfields = [semT[j:j + 1, :] for j in range(semT.shape[0])] + [u_ref[...]]
    B_ref[...] = jnp.concatenate(
        [semv_ref[...], ucol_ref[...], jnp.ones_like(ucol_ref[...])],
        axis=1).astype(jnp.bfloat16)

    rflat = [rcol[k][i] for k in range(3) for i in range(3)]
    words = []
    for j in range(4):
        words.append(packpair(rflat[2 * j], rflat[2 * j + 1]))
    words.append(packpair(rflat[8], jnp.zeros_like(rflat[8])))
    for j in range(9):
        words.append(packpair(bfields[2 * j], bfields[2 * j + 1]))
    for k in range(3):
        words.append(bits(wks[k]))
    for k in range(3):
        words.append(bits(cks[k]))
    words.append(bits(opas_ref[...]))
    attrs_ref[...] = jnp.concatenate(words, axis=0)

    mintT = mintT_ref[...]
    radii = radii_ref[...]
    attrsi_ref[...] = ((radii << 24) | (mintT[0:1, :] << 16)
                       | (mintT[1:2, :] << 8) | mintT[2:3, :])

    gx = (mT[0:1, :] * float(_CPA) + 1.0).astype(i32) - 1
    gy = (mT[1:2, :] * float(_CPA) + 1.0).astype(i32) - 1
    gz = (mT[2:3, :] * float(_CPA) + 1.0).astype(i32) - 1
    cell = (gx * _CPA + gy) * _CPA + gz

    mc = means_ref[...]
    gxc = (mc[:, 0:1] * float(_CPA) + 1.0).astype(i32) - 1
    gyc = (mc[:, 1:2] * float(_CPA) + 1.0).astype(i32) - 1
    gzc = (mc[:, 2:3] * float(_CPA) + 1.0).astype(i32) - 1
    cellcol = (gxc * _CPA + gyc) * _CPA + gzc

    m = cell.shape[1]
    cells_iota = lax.broadcasted_iota(i32, (_NCELLS, m), 0)
    A = (cells_iota == cell).astype(f32)
    counts = jnp.sum(A, axis=1, keepdims=True)
    counts_ref[...] = jnp.minimum(counts, float(_CAP)).astype(i32)
    eq = cellcol == cell
    mi = lax.broadcasted_iota(i32, (m, m), 0)
    mj = lax.broadcasted_iota(i32, (m, m), 1)
    E = (eq & (mj < mi)).astype(f32)
    rank = jnp.sum(E, axis=1, keepdims=True).astype(i32)
    k_iota = lax.broadcasted_iota(i32, (m, _CAP), 1)
    gid1 = (lax.broadcasted_iota(i32, (m, _CAP), 0) + 1).astype(f32)
    W = jnp.where(rank == k_iota, gid1, 0.0)
    slots = jax.lax.dot_general(
        A, W, (((1,), (0,)), ((), ())),
        precision=jax.lax.Precision.HIGHEST,
        preferred_element_type=f32)
    slots_ref[...] = slots.astype(i32)


def _agg_kernel(pts_ref, pint_ref, r0_ref, r1_ref, r2_ref, ck_ref, w_ref,
                opas_ref, mint_ref, radii_ref, B_ref, out_ref):
    pts = pts_ref[...]
    ck = ck_ref[...]
    w = w_ref[...]

    power = None
    for k, r_ref in enumerate((r0_ref, r1_ref, r2_ref)):
        Pk = jnp.dot(pts, r_ref[...], preferred_element_type=jnp.float32)
        dd = Pk - ck[k:k + 1, :]
        term = dd * dd * w[k:k + 1, :]
        power = term if power is None else power + term

    pint = pint_ref[...]
    mint = mint_ref[...]
    radii = radii_ref[...]
    mask = (
        (jnp.abs(pint[:, 0:1] - mint[0:1, :]) <= radii)
        & (jnp.abs(pint[:, 1:2] - mint[1:2, :]) <= radii)
        & (jnp.abs(pint[:, 2:3] - mint[2:3, :]) <= radii)
    )

    g = jnp.exp(-0.5 * power) * opas_ref[...]
    a = jnp.where(mask, g, 0.0).astype(jnp.bfloat16)
    out_ref[...] = jnp.dot(a, B_ref[...], preferred_element_type=jnp.float32)


def _bf16_round(t):
    yv = plsc.bitcast(t, jnp.int32)
    r = (yv + 0x7FFF + ((yv >> 16) & 1)) & ~0xFFFF
    return plsc.bitcast(r, jnp.float32)


def _hi(word):
    return plsc.bitcast(word & ~0xFFFF, jnp.float32)


def _lo(word):
    return plsc.bitcast(word << 16, jnp.float32)


def _sc_body(m, chunk, attrs_hbm, attrsi_hbm, counts_hbm, slots_hbm,
             px_hbm, py_hbm, pz_hbm, ppk_hbm, out_hbm,
             attrs_v, attrsi_v, counts_v, slots_v,
             px_v, py_v, pz_v, ppk_v, out_v):
    wid = lax.axis_index("s") * 2 + lax.axis_index("c")
    base = wid * chunk
    pltpu.sync_copy(attrs_hbm, attrs_v)
    pltpu.sync_copy(attrsi_hbm, attrsi_v)
    pltpu.sync_copy(counts_hbm, counts_v)
    pltpu.sync_copy(slots_hbm, slots_v)
    pltpu.sync_copy(px_hbm.at[pl.ds(base, chunk)], px_v)
    pltpu.sync_copy(py_hbm.at[pl.ds(base, chunk)], py_v)
    pltpu.sync_copy(pz_hbm.at[pl.ds(base, chunk)], pz_v)
    pltpu.sync_copy(ppk_hbm.at[pl.ds(base, chunk)], ppk_v)

    zeros16 = jnp.zeros((16,), jnp.float32)

    def zero_body(i, carry):
        out_v[pl.ds(i * 16, 16)] = zeros16
        return carry

    lax.fori_loop(0, chunk * _OSTRIDE // 16, zero_body, 0)

    lane = lax.iota(jnp.int32, 16)

    def vreg_body(i, carry):
        sl = pl.ds(i * 16, 16)
        px = px_v[sl]
        py = py_v[sl]
        pz = pz_v[sl]
        ppk = ppk_v[sl]
        pxi = ppk >> 20
        pyi = (ppk >> 10) & 1023
        pzi = ppk & 1023
        obase = (i * 16 + lane) * _OSTRIDE

        los, his = [], []
        for coord in (px, py, pz):
            lo_f = (coord - _MARGIN) * float(_CPA)
            hi_f = (coord + _MARGIN) * float(_CPA)
            los.append((lo_f + 1.0).astype(jnp.int32) - 1)
            his.append((hi_f + 1.0).astype(jnp.int32) - 1)

        def visit(cell, valid):
            cellc = jnp.where(valid, cell, 0)
            cnt = plsc.load_gather(counts_v, [cellc], mask=valid)
            cnt = jnp.where(valid, cnt, 0)
            mx = jnp.max(cnt)

            def jbody(j, jcarry):
                act = j < cnt
                sidx = jnp.where(act, cellc * _CAP + j, 0)
                gid1 = plsc.load_gather(slots_v, [sidx], mask=act)
                act2 = act & (gid1 > 0)
                gid = jnp.where(act2, gid1 - 1, 0)
                pk = plsc.load_gather(attrsi_v, [gid], mask=act2)
                rad = pk >> 24
                mxc = (pk >> 16) & 255
                myc = (pk >> 8) & 255
                mzc = pk & 255
                hit = (act2
                       & (jnp.abs(pxi - mxc) <= rad)
                       & (jnp.abs(pyi - myc) <= rad)
                       & (jnp.abs(pzi - mzc) <= rad))
                any_hit = jnp.max(hit.astype(jnp.int32))

                @pl.when(any_hit > 0)
                def _eval():
                    rw = [plsc.load_gather(attrs_v, [j * m + gid], mask=hit)
                          for j in range(5)]
                    r00, r01 = _hi(rw[0]), _lo(rw[0])
                    r02, r10 = _hi(rw[1]), _lo(rw[1])
                    r11, r12 = _hi(rw[2]), _lo(rw[2])
                    r20, r21 = _hi(rw[3]), _lo(rw[3])
                    r22 = _hi(rw[4])
                    rc = ((r00, r01, r02), (r10, r11, r12), (r20, r21, r22))
                    power = None
                    for k in range(3):
                        wk = plsc.bitcast(
                            plsc.load_gather(attrs_v, [(14 + k) * m + gid], mask=hit),
                            jnp.float32)
                        ck = plsc.bitcast(
                            plsc.load_gather(attrs_v, [(17 + k) * m + gid], mask=hit),
                            jnp.float32)
                        dd = px * rc[k][0] + py * rc[k][1] + pz * rc[k][2] - ck
                        t = dd * dd * wk
                        power = t if power is None else power + t
                    opa = plsc.bitcast(
                        plsc.load_gather(attrs_v, [20 * m + gid], mask=hit),
                        jnp.float32)
                    a = jnp.exp(power * -0.5) * opa
                    a_r = _bf16_round(jnp.where(hit, a, 0.0))
                    plsc.addupdate_scatter(out_v, [obase + 18], a_r, mask=hit)
                    for j in range(9):
                        bw = plsc.load_gather(attrs_v, [(5 + j) * m + gid], mask=hit)
                        plsc.addupdate_scatter(out_v, [obase + 2 * j],
                                               a_r * _hi(bw), mask=hit)
                        plsc.addupdate_scatter(out_v, [obase + 2 * j + 1],
                                               a_r * _lo(bw), mask=hit)

                return jcarry

            lax.fori_loop(0, mx, jbody, 0)

        for dx in range(2):
            cx = los[0] + dx
            mvx = (cx >= 0) & (cx <= _CPA - 1)
            if dx == 1:
                mvx = mvx & (his[0] >= cx)
            for dy in range(2):
                cy = los[1] + dy
                mvy = mvx & (cy >= 0) & (cy <= _CPA - 1)
                if dy == 1:
                    mvy = mvy & (his[1] >= cy)
                for dz in range(2):
                    cz = los[2] + dz
                    mvz = mvy & (cz >= 0) & (cz <= _CPA - 1)
                    if dz == 1:
                        mvz = mvz & (his[2] >= cz)
                    visit((cx * _CPA + cy) * _CPA + cz, mvz)
        return carry

    lax.fori_loop(0, chunk // 16, vreg_body, 0)
    pltpu.sync_copy(out_v, out_hbm.at[pl.ds(base * _OSTRIDE, chunk * _OSTRIDE)])


def kernel(pts, means3D, opas, u, v, semantics, scales, rot3D):
    pts = pts[0]
    means3D = means3D[0]
    opas = opas[0]
    u = u[0]
    v = v[0]
    semantics = semantics[0]
    scales = scales[0]
    rot3D = rot3D[0]

    n, m = pts.shape[0], means3D.shape[0]
    c = semantics.shape[1]

    ns = n - _NT
    chunk = -(-ns // (_NW * 16)) * 16
    npad = _NW * chunk

    semv = semantics * v[:, None]
    pint = jnp.floor(pts / _GRID).astype(jnp.int32)
    mintT = jnp.floor(means3D / _GRID).astype(jnp.int32).T
    radii = jnp.maximum(
        jnp.ceil(scales.max(axis=-1) * _SCALE_MULT / _GRID), _RADII_MIN
    ).astype(jnp.int32)[None, :]

    (attrs, attrsi, counts, slots,
     r0T, r1T, r2T, cks, ws, B) = pl.pallas_call(
        _prep_kernel,
        out_shape=[
            jax.ShapeDtypeStruct((_NF, m), jnp.int32),
            jax.ShapeDtypeStruct((1, m), jnp.int32),
            jax.ShapeDtypeStruct((_NCELLS, 1), jnp.int32),
            jax.ShapeDtypeStruct((_NCELLS, _CAP), jnp.int32),
            jax.ShapeDtypeStruct((3, m), jnp.bfloat16),
            jax.ShapeDtypeStruct((3, m), jnp.bfloat16),
            jax.ShapeDtypeStruct((3, m), jnp.bfloat16),
            jax.ShapeDtypeStruct((3, m), jnp.float32),
            jax.ShapeDtypeStruct((3, m), jnp.float32),
            jax.ShapeDtypeStruct((m, c + 2), jnp.bfloat16),
        ],
    )(means3D, means3D.T, rot3D.T, scales.T, semv.T, semv, u[:, None],
      opas[None, :], u[None, :], mintT, radii)

    rnd = lambda t: t.astype(jnp.bfloat16).astype(jnp.float32)
    pad = npad - ns
    pts_s = pts[_NT:]
    pint_s = pint[_NT:]
    pxr = jnp.pad(rnd(pts_s[:, 0]), (0, pad), constant_values=10.0)
    pyr = jnp.pad(rnd(pts_s[:, 1]), (0, pad), constant_values=10.0)
    pzr = jnp.pad(rnd(pts_s[:, 2]), (0, pad), constant_values=10.0)
    ppk = jnp.pad((pint_s[:, 0] << 20) | (pint_s[:, 1] << 10) | pint_s[:, 2],
                  (0, pad))

    sc = pl.kernel(
        functools.partial(_sc_body, m, chunk),
        out_type=jax.ShapeDtypeStruct((npad * _OSTRIDE,), jnp.float32),
        mesh=plsc.VectorSubcoreMesh(core_axis_name="c", subcore_axis_name="s"),
        compiler_params=pltpu.CompilerParams(needs_layout_passes=False),
        scratch_types=[
            pltpu.VMEM((_NF * m,), jnp.int32),
            pltpu.VMEM((m,), jnp.int32),
            pltpu.VMEM((_NCELLS,), jnp.int32),
            pltpu.VMEM((_NCELLS * _CAP,), jnp.int32),
            pltpu.VMEM((chunk,), jnp.float32),
            pltpu.VMEM((chunk,), jnp.float32),
            pltpu.VMEM((chunk,), jnp.float32),
            pltpu.VMEM((chunk,), jnp.int32),
            pltpu.VMEM((chunk * _OSTRIDE,), jnp.float32),
        ],
    )
    out_s = sc(attrs.reshape(-1), attrsi.reshape(-1), counts.reshape(-1),
               slots.reshape(-1), pxr, pyr, pzr, ppk)
    out_s = out_s.reshape(npad, _OSTRIDE)

    grid = _NT // _TC_BLK
    full = lambda i: (0, 0)
    out_d = pl.pallas_call(
        _agg_kernel,
        grid=(grid,),
        in_specs=[
            pl.BlockSpec((_TC_BLK, 3), lambda i: (i, 0)),
            pl.BlockSpec((_TC_BLK, 3), lambda i: (i, 0)),
            pl.BlockSpec((3, m), full),
            pl.BlockSpec((3, m), full),
            pl.BlockSpec((3, m), full),
            pl.BlockSpec((3, m), full),
            pl.BlockSpec((3, m), full),
            pl.BlockSpec((1, m), full),
            pl.BlockSpec((3, m), full),
            pl.BlockSpec((1, m), full),
            pl.BlockSpec((m, c + 2), full),
        ],
        out_specs=pl.BlockSpec((_TC_BLK, c + 2), lambda i: (i, 0)),
        out_shape=jax.ShapeDtypeStruct((_NT, c + 2), jnp.float32),
        compiler_params=pltpu.CompilerParams(
            dimension_semantics=("arbitrary",),
        ),
    )(pts[:_NT].astype(jnp.bfloat16), pint[:_NT], r0T, r1T, r2T,
      cks, ws, opas[None, :], mintT, radii, B)

    logits = jnp.concatenate([out_d[:, :c], out_s[:ns, :c]], axis=0)
    bin_logits = jnp.concatenate([out_d[:, c], out_s[:ns, c]], axis=0)
    density = jnp.concatenate([out_d[:, c + 1], out_s[:ns, c + 1]], axis=0)
    return logits, bin_logits, density
